# baseline (speedup 1.0000x reference)
import jax
import jax.numpy as jnp
from jax import lax
from jax.experimental import pallas as pl
from jax.experimental.pallas import tpu as pltpu
from jax.experimental.pallas import tpu_sc as plsc

_B, _F, _D = 16384, 26, 64
_N = _B * _F
_OD = 2 * _D
_NC, _NS, _L = 2, 16, 16
_NW = _NC * _NS
_PER_W = _N // _NW
_CH = 128
_NSUPER = _PER_W // (2 * _CH)
_RPT = 312
_LROWS = _NS * _RPT


def _popcount16(x):
    v = x - ((x >> 1) & 0x5555)
    v = (v & 0x3333) + ((v >> 2) & 0x3333)
    v = (v + (v >> 4)) & 0x0F0F
    return (v + (v >> 8)) & 0x1F


def _div3(y):
    return lax.shift_right_logical(y * 43691, 17)


def _sc_body(x_hbm, e2_hbm, e3_hbm, out_hbm,
             xv, e2v, e3v, stag, idx_a, idx_b, obuf_a, obuf_b, lut_sh,
             sem_ga, sem_gb, sem_oa, sem_ob):
    cid = lax.axis_index("c")
    sid = lax.axis_index("s")
    wid = sid * _NC + cid
    base = wid * _PER_W
    pltpu.sync_copy(x_hbm.at[pl.ds(base, _PER_W)], xv)
    pltpu.sync_copy(e2_hbm, e2v)
    pltpu.sync_copy(e3_hbm, e3v)

    a2, d2, a3, d31, d32 = [], [], [], [], []
    for j in range(4):
        r0 = e2v[pl.ds(j * _L, _L)]
        r1 = e2v[pl.ds(_D + j * _L, _L)]
        a2.append(16.0 * r0)
        d2.append(r1 - r0)
        s0 = e3v[pl.ds(j * _L, _L)]
        s1 = e3v[pl.ds(_D + j * _L, _L)]
        s2 = e3v[pl.ds(2 * _D + j * _L, _L)]
        a3.append(16.0 * s0)
        d31.append(s1 - s0)
        d32.append(s2 - s0)

    def lbody(k, _):
        r = sid * _RPT + k
        p = lax.shift_right_logical(r * 58053, 24)
        rem = r - p * 289
        c1 = lax.shift_right_logical(rem * 61681, 20)
        c2 = rem - c1 * 17
        pf = p.astype(jnp.float32)
        c1f = c1.astype(jnp.float32)
        c2f = c2.astype(jnp.float32)
        for j in range(4):
            stag[k, pl.ds(j * _L, _L)] = a2[j] + pf * d2[j]
            stag[k, pl.ds(_D + j * _L, _L)] = a3[j] + c1f * d31[j] + c2f * d32[j]
        return 0

    lax.fori_loop(0, _RPT, lbody, 0)
    pltpu.sync_copy(stag, lut_sh.at[pl.ds(sid * _RPT, _RPT)])
    plsc.subcore_barrier()

    def idx_for_chunk(chunk_off, idxbuf):
        def cbody(v, _):
            x = xv[pl.ds(chunk_off + v * _L, _L)]
            p = _popcount16(x)
            s = x - x
            n2 = x - x
            y = x
            for _i in range(11):
                q = _div3(y)
                d = y - (q + (q << 1))
                s = s + d
                n2 = n2 + (d >> 1)
                y = q
            n1 = s - (n2 << 1)
            idxbuf[pl.ds(v * _L, _L)] = p * 289 + n1 * 17 + n2
            return 0
        lax.fori_loop(0, _CH // _L, cbody, 0)

    def do_chunk(s, which, idxbuf, obuf, sem_g, sem_o):
        c0 = (2 * s + which) * _CH

        @pl.when(s > 0)
        def _wait_prev():
            pltpu.make_async_copy(
                obuf, out_hbm.at[pl.ds(0, _CH)], sem_o).wait()

        idx_for_chunk(c0, idxbuf)
        pltpu.async_copy(lut_sh.at[idxbuf], obuf, sem_g).wait()
        pltpu.make_async_copy(
            obuf, out_hbm.at[pl.ds(base + c0, _CH)], sem_o).start()

    def sbody(s, _):
        do_chunk(s, 0, idx_a, obuf_a, sem_ga, sem_oa)
        do_chunk(s, 1, idx_b, obuf_b, sem_gb, sem_ob)
        return 0

    lax.fori_loop(0, _NSUPER, sbody, 0)
    pltpu.make_async_copy(obuf_a, out_hbm.at[pl.ds(0, _CH)], sem_oa).wait()
    pltpu.make_async_copy(obuf_b, out_hbm.at[pl.ds(0, _CH)], sem_ob).wait()


@jax.jit
def kernel(input, emb2, emb3):
    run = pl.kernel(
        _sc_body,
        out_type=jax.ShapeDtypeStruct((_N, _OD), jnp.float32),
        mesh=plsc.VectorSubcoreMesh(core_axis_name="c", subcore_axis_name="s"),
        scratch_types=[
            pltpu.VMEM((_PER_W,), jnp.int32),
            pltpu.VMEM((_OD,), jnp.float32),
            pltpu.VMEM((3 * _D,), jnp.float32),
            pltpu.VMEM((_RPT, _OD), jnp.float32),
            pltpu.VMEM((_CH,), jnp.int32),
            pltpu.VMEM((_CH,), jnp.int32),
            pltpu.VMEM((_CH, _OD), jnp.float32),
            pltpu.VMEM((_CH, _OD), jnp.float32),
            pltpu.VMEM_SHARED((_LROWS, _OD), jnp.float32),
            pltpu.SemaphoreType.DMA,
            pltpu.SemaphoreType.DMA,
            pltpu.SemaphoreType.DMA,
            pltpu.SemaphoreType.DMA,
        ],
    )
    out = run(input.reshape(_N), emb2.reshape(_OD), emb3.reshape(3 * _D))
    return out.reshape(_B, _F, _OD)


# speedup vs baseline: 1.5926x; 1.5926x over previous
import jax
import jax.numpy as jnp
from jax import lax
from jax.experimental import pallas as pl
from jax.experimental.pallas import tpu as pltpu
from jax.experimental.pallas import tpu_sc as plsc

_B, _F, _D = 16384, 26, 64
_FP = 32
_N = _B * _F
_OD = 2 * _D
_NC, _NS, _L = 2, 16, 16
_NW = _NC * _NS
_BPW = _B // _NW
_CB = 8
_CR = _CB * _FP
_NSUPER = _BPW // (2 * _CB)
_RPT = 320
_LROWS = _NS * _RPT


def _popcount16(x):
    v = x - ((x >> 1) & 0x5555)
    v = (v & 0x3333) + ((v >> 2) & 0x3333)
    v = (v + (v >> 4)) & 0x0F0F
    return (v + (v >> 8)) & 0x1F


def _div3(y):
    return lax.shift_right_logical(y * 43691, 17)


def _sc_body(x_hbm, e2_hbm, e3_hbm, out_hbm,
             xv, e2v, e3v, idx_ah, idx_al, idx_bh, idx_bl,
             obuf_a, obuf_b, lut_sh, sem_ga, sem_gb, sem_oa, sem_ob):
    cid = lax.axis_index("c")
    sid = lax.axis_index("s")
    wid = sid * _NC + cid
    ebase = wid * _BPW * _F
    rbase = wid * _BPW * _FP
    pltpu.sync_copy(x_hbm.at[pl.ds(ebase, _BPW * _F)],
                    xv.at[pl.ds(0, _BPW * _F)])
    pltpu.sync_copy(e2_hbm, e2v)
    pltpu.sync_copy(e3_hbm, e3v)

    a2, d2, a3, d31, d32 = [], [], [], [], []
    for j in range(4):
        r0 = e2v[pl.ds(j * _L, _L)]
        r1 = e2v[pl.ds(_D + j * _L, _L)]
        a2.append(16.0 * r0)
        d2.append(r1 - r0)
        s0 = e3v[pl.ds(j * _L, _L)]
        s1 = e3v[pl.ds(_D + j * _L, _L)]
        s2 = e3v[pl.ds(2 * _D + j * _L, _L)]
        a3.append(16.0 * s0)
        d31.append(s1 - s0)
        d32.append(s2 - s0)

    def build_lut(off):
        def lbody(k, _):
            r = sid * _RPT + off + k
            p = lax.shift_right_logical(r * 58053, 24)
            rem = r - p * 289
            c1 = lax.shift_right_logical(rem * 61681, 20)
            c2 = rem - c1 * 17
            pf = p.astype(jnp.float32)
            c1f = c1.astype(jnp.float32)
            c2f = c2.astype(jnp.float32)
            for j in range(4):
                obuf_a[k, pl.ds(j * _L, _L)] = a2[j] + pf * d2[j]
                obuf_a[k, pl.ds(_D + j * _L, _L)] = (
                    a3[j] + c1f * d31[j] + c2f * d32[j])
            return 0
        lax.fori_loop(0, _RPT // 2, lbody, 0)
        pltpu.sync_copy(obuf_a.at[pl.ds(0, _RPT // 2)],
                        lut_sh.at[pl.ds(sid * _RPT + off, _RPT // 2)])

    build_lut(0)
    build_lut(_RPT // 2)
    plsc.subcore_barrier()

    def idx_for_chunk(chunk_e0, idxh, idxl):
        def cbody_for(idxbuf, voff):
            def cbody(v, _):
                vv = voff + v
                b = vv >> 1
                f0 = (vv & 1) * _L
                x = xv[pl.ds(chunk_e0 + b * _F + f0, _L)] & 0xFFFF
                p = _popcount16(x)
                s = x - x
                n2 = x - x
                y = x
                for _i in range(11):
                    q = _div3(y)
                    d = y - (q + (q << 1))
                    s = s + d
                    n2 = n2 + (d >> 1)
                    y = q
                n1 = s - (n2 << 1)
                idxbuf[pl.ds(v * _L, _L)] = p * 289 + n1 * 17 + n2
                return 0
            lax.fori_loop(0, _CR // 2 // _L, cbody, 0)
        cbody_for(idxh, 0)
        cbody_for(idxl, _CR // 2 // _L)

    def do_chunk(s, which, idxh, idxl, obuf, sem_g, sem_o):
        cb0 = (2 * s + which) * _CB

        @pl.when(s > 0)
        def _wait_prev():
            pltpu.make_async_copy(
                obuf, out_hbm.at[pl.ds(0, _CR)], sem_o).wait()

        idx_for_chunk(cb0 * _F, idxh, idxl)
        g1 = pltpu.async_copy(lut_sh.at[idxh], obuf.at[pl.ds(0, _CR // 2)],
                              sem_g)
        g2 = pltpu.async_copy(lut_sh.at[idxl], obuf.at[pl.ds(_CR // 2, _CR // 2)],
                              sem_g)
        g1.wait()
        g2.wait()
        pltpu.make_async_copy(
            obuf, out_hbm.at[pl.ds(rbase + cb0 * _FP, _CR)], sem_o).start()

    def sbody(s, _):
        do_chunk(s, 0, idx_ah, idx_al, obuf_a, sem_ga, sem_oa)
        do_chunk(s, 1, idx_bh, idx_bl, obuf_b, sem_gb, sem_ob)
        return 0

    lax.fori_loop(0, _NSUPER, sbody, 0)
    pltpu.make_async_copy(obuf_a, out_hbm.at[pl.ds(0, _CR)], sem_oa).wait()
    pltpu.make_async_copy(obuf_b, out_hbm.at[pl.ds(0, _CR)], sem_ob).wait()


@jax.jit
def kernel(input, emb2, emb3):
    run = pl.kernel(
        _sc_body,
        out_type=jax.ShapeDtypeStruct((_B * _FP, _OD), jnp.float32),
        mesh=plsc.VectorSubcoreMesh(core_axis_name="c", subcore_axis_name="s"),
        scratch_types=[
            pltpu.VMEM((_BPW * _F + _L,), jnp.int32),
            pltpu.VMEM((_OD,), jnp.float32),
            pltpu.VMEM((3 * _D,), jnp.float32),
            pltpu.VMEM((_CR // 2,), jnp.int32),
            pltpu.VMEM((_CR // 2,), jnp.int32),
            pltpu.VMEM((_CR // 2,), jnp.int32),
            pltpu.VMEM((_CR // 2,), jnp.int32),
            pltpu.VMEM((_CR, _OD), jnp.float32),
            pltpu.VMEM((_CR, _OD), jnp.float32),
            pltpu.VMEM_SHARED((_LROWS, _OD), jnp.float32),
            pltpu.SemaphoreType.DMA,
            pltpu.SemaphoreType.DMA,
            pltpu.SemaphoreType.DMA,
            pltpu.SemaphoreType.DMA,
        ],
    )
    out = run(input.reshape(_N), emb2.reshape(_OD), emb3.reshape(3 * _D))
    return out.reshape(_B, _FP, _OD)[:, :_F, :]


# speedup vs baseline: 1.6437x; 1.0321x over previous
import jax
import jax.numpy as jnp
from jax import lax
from jax.experimental import pallas as pl
from jax.experimental.pallas import tpu as pltpu
from jax.experimental.pallas import tpu_sc as plsc

_B, _F, _D = 16384, 26, 64
_FP = 32
_N = _B * _F
_OD = 2 * _D
_NC, _NS, _L = 2, 16, 16
_NW = _NC * _NS
_BPW = _B // _NW
_CB = 8
_CR = _CB * _FP
_NSUPER = _BPW // (2 * _CB)
_RPT = 320
_LROWS = _NS * _RPT


def _popcount16(x):
    v = x - ((x >> 1) & 0x5555)
    v = (v & 0x3333) + ((v >> 2) & 0x3333)
    v = (v + (v >> 4)) & 0x0F0F
    return (v + (v >> 8)) & 0x1F


def _div3(y):
    return lax.shift_right_logical(y * 43691, 17)


def _sc_body(x_hbm, e2_hbm, e3_hbm, out_hbm,
             xv, e2v, e3v, idx_ah, idx_al, idx_bh, idx_bl,
             obuf_a, obuf_b, lut_sh, sem_ga, sem_gb, sem_oa, sem_ob):
    cid = lax.axis_index("c")
    sid = lax.axis_index("s")
    wid = sid * _NC + cid
    ebase = wid * _BPW * _F
    rbase = wid * _BPW * _FP
    pltpu.sync_copy(x_hbm.at[pl.ds(ebase, _BPW * _F)],
                    xv.at[pl.ds(0, _BPW * _F)])
    pltpu.sync_copy(e2_hbm, e2v)
    pltpu.sync_copy(e3_hbm, e3v)

    a2, d2, a3, d31, d32 = [], [], [], [], []
    for j in range(4):
        r0 = e2v[pl.ds(j * _L, _L)]
        r1 = e2v[pl.ds(_D + j * _L, _L)]
        a2.append(16.0 * r0)
        d2.append(r1 - r0)
        s0 = e3v[pl.ds(j * _L, _L)]
        s1 = e3v[pl.ds(_D + j * _L, _L)]
        s2 = e3v[pl.ds(2 * _D + j * _L, _L)]
        a3.append(16.0 * s0)
        d31.append(s1 - s0)
        d32.append(s2 - s0)

    def build_lut(off):
        def lbody(k, _):
            r = sid * _RPT + off + k
            p = lax.shift_right_logical(r * 58053, 24)
            rem = r - p * 289
            c1 = lax.shift_right_logical(rem * 61681, 20)
            c2 = rem - c1 * 17
            pf = p.astype(jnp.float32)
            c1f = c1.astype(jnp.float32)
            c2f = c2.astype(jnp.float32)
            for j in range(4):
                obuf_a[k, pl.ds(j * _L, _L)] = a2[j] + pf * d2[j]
                obuf_a[k, pl.ds(_D + j * _L, _L)] = (
                    a3[j] + c1f * d31[j] + c2f * d32[j])
            return 0
        lax.fori_loop(0, _RPT // 2, lbody, 0)
        pltpu.sync_copy(obuf_a.at[pl.ds(0, _RPT // 2)],
                        lut_sh.at[pl.ds(sid * _RPT + off, _RPT // 2)])

    build_lut(0)
    build_lut(_RPT // 2)
    plsc.subcore_barrier()

    def cbody_half(chunk_e0, idxbuf, voff):
        def cbody(v, _):
            vv = voff + v
            b = vv >> 1
            f0 = (vv & 1) * _L
            x = xv[pl.ds(chunk_e0 + b * _F + f0, _L)] & 0xFFFF
            p = _popcount16(x)
            s = x - x
            n2 = x - x
            y = x
            for _i in range(11):
                q = _div3(y)
                d = y - (q + (q << 1))
                s = s + d
                n2 = n2 + (d >> 1)
                y = q
            n1 = s - (n2 << 1)
            idxbuf[pl.ds(v * _L, _L)] = p * 289 + n1 * 17 + n2
            return 0
        lax.fori_loop(0, _CR // 2 // _L, cbody, 0)

    def do_chunk(s, which, idxh, idxl, obuf, sem_g, sem_o):
        cb0 = (2 * s + which) * _CB

        @pl.when(s > 0)
        def _wait_prev():
            pltpu.make_async_copy(
                obuf, out_hbm.at[pl.ds(0, _CR)], sem_o).wait()

        cbody_half(cb0 * _F, idxh, 0)
        g1 = pltpu.async_copy(lut_sh.at[idxh], obuf.at[pl.ds(0, _CR // 2)],
                              sem_g)
        cbody_half(cb0 * _F, idxl, _CR // 2 // _L)
        g2 = pltpu.async_copy(lut_sh.at[idxl], obuf.at[pl.ds(_CR // 2, _CR // 2)],
                              sem_g)
        g1.wait()
        g2.wait()
        pltpu.make_async_copy(
            obuf, out_hbm.at[pl.ds(rbase + cb0 * _FP, _CR)], sem_o).start()

    def sbody(s, _):
        do_chunk(s, 0, idx_ah, idx_al, obuf_a, sem_ga, sem_oa)
        do_chunk(s, 1, idx_bh, idx_bl, obuf_b, sem_gb, sem_ob)
        return 0

    lax.fori_loop(0, _NSUPER, sbody, 0)
    pltpu.make_async_copy(obuf_a, out_hbm.at[pl.ds(0, _CR)], sem_oa).wait()
    pltpu.make_async_copy(obuf_b, out_hbm.at[pl.ds(0, _CR)], sem_ob).wait()


@jax.jit
def kernel(input, emb2, emb3):
    run = pl.kernel(
        _sc_body,
        out_type=jax.ShapeDtypeStruct((_B * _FP, _OD), jnp.float32),
        mesh=plsc.VectorSubcoreMesh(core_axis_name="c", subcore_axis_name="s"),
        scratch_types=[
            pltpu.VMEM((_BPW * _F + _L,), jnp.int32),
            pltpu.VMEM((_OD,), jnp.float32),
            pltpu.VMEM((3 * _D,), jnp.float32),
            pltpu.VMEM((_CR // 2,), jnp.int32),
            pltpu.VMEM((_CR // 2,), jnp.int32),
            pltpu.VMEM((_CR // 2,), jnp.int32),
            pltpu.VMEM((_CR // 2,), jnp.int32),
            pltpu.VMEM((_CR, _OD), jnp.float32),
            pltpu.VMEM((_CR, _OD), jnp.float32),
            pltpu.VMEM_SHARED((_LROWS, _OD), jnp.float32),
            pltpu.SemaphoreType.DMA,
            pltpu.SemaphoreType.DMA,
            pltpu.SemaphoreType.DMA,
            pltpu.SemaphoreType.DMA,
        ],
    )
    out = run(input.reshape(_N), emb2.reshape(_OD), emb3.reshape(3 * _D))
    return out.reshape(_B, _FP, _OD)[:, :_F, :]
